# Initial kernel scaffold; baseline (speedup 1.0000x reference)
import jax
import jax.numpy as jnp
from jax.experimental import pallas as pl


def kernel(x, edge_index, gat_wl, gat_bl, gat_wr, gat_br, gat_att, gat_bias, ln1_g, ln1_b, wq, wk, wv, wo, bo, proj, ln2_g, ln2_b, ff_w1, ff_b1, ff_w2, ff_b2, lnf_g, lnf_b):
    raise NotImplementedError("write your pallas kernel here")


# speedup vs baseline: 12.2211x; 12.2211x over previous
import functools

import jax
import jax.numpy as jnp
from jax import lax
from jax.experimental import pallas as pl
from jax.experimental.pallas import tpu as pltpu
from jax.experimental.pallas import tpu_sc as plsc

_B, _G, _D, _H = 4, 10000, 320, 4
_DH = _D // _H
_E = 160000
_M = 256
_FF = 4 * _D
_L = 2
_BG = _B * _G
_RATIO = _M ** -0.5
_DN = _DH ** -0.25

_NC, _NS = 2, 16
_EPT = _E // _NS
_C = 80
_NCH = _EPT // _C
_DW = 2 * _BG // _NS

_f32 = jnp.float32
_i32 = jnp.int32


def _hsum16(v, lane):
    for k in (8, 4, 2, 1):
        v = v + jnp.take_along_axis(v, lane ^ k, axis=0)
    return v


def _ln(x, g, b):
    mu = jnp.mean(x, axis=-1, keepdims=True)
    xc = x - mu
    var = jnp.mean(xc * xc, axis=-1, keepdims=True)
    return xc * lax.rsqrt(var + 1e-5) * g + b


def _k0_body(x_ref, w_ref, b_ref, o_ref):
    res = jnp.dot(x_ref[...], w_ref[...], preferred_element_type=_f32)
    res = res + b_ref[0]
    o_ref[0] = res[:, 0:160]
    o_ref[1] = res[:, 320:480]
    o_ref[2] = res[:, 160:320]
    o_ref[3] = res[:, 480:640]


def _k0(xf, wlr, blr):
    t0 = 2000
    return pl.pallas_call(
        _k0_body,
        grid=(_BG // t0,),
        in_specs=[
            pl.BlockSpec((t0, _D), lambda t: (t, 0)),
            pl.BlockSpec((_D, 2 * _D), lambda t: (0, 0)),
            pl.BlockSpec((1, 2 * _D), lambda t: (0, 0)),
        ],
        out_specs=pl.BlockSpec((4, t0, 160), lambda t: (0, t, 0)),
        out_shape=jax.ShapeDtypeStruct((4, _BG, 160), _f32),
        compiler_params=pltpu.CompilerParams(
            dimension_semantics=("arbitrary",)),
    )(xf, wlr, blr)


def _p1_body(tab, src_h, dst_h, att2, ex_out, den_out,
             rows_l, rows_r, glidx, gridx, exbuf, denidx, attv, zstage,
             den_sh, sem_l, sem_r):
    c = lax.axis_index("c")
    s = lax.axis_index("s")
    lane = lax.iota(_i32, 16)
    half = lax.shift_right_logical(lane, 1)
    bit = lane & 1
    gl_off = (2 * c) * _BG
    gr_off = gl_off + _BG

    zv = jnp.zeros((16,), _f32)

    @pl.loop(0, 64)
    def _zfill(i):
        zstage[pl.ds(i * 16, 16)] = zv

    for i in range(5):
        pltpu.sync_copy(zstage.at[pl.ds(0, 1000)],
                        den_sh.at[pl.ds(s * _DW + i * 1000, 1000)])

    pltpu.sync_copy(att2.at[c], attv)
    plsc.subcore_barrier()

    attregs = [attv[0, pl.ds(16 * j, 16)] for j in range(10)]

    @pl.loop(0, _B)
    def _batch(b):
        ex_base = (c * _B + b) * _E + s * _EPT
        node_l = gl_off + b * _G
        node_r = gr_off + b * _G
        den_off = b * _G

        @pl.loop(0, _NCH)
        def _chunk(k):
            eb = s * _EPT + k * _C
            pltpu.sync_copy(src_h.at[pl.ds(eb, _C)], glidx)
            pltpu.sync_copy(dst_h.at[pl.ds(eb, _C)], gridx)

            @pl.loop(0, _C // 16)
            def _ix(i):
                sl = pl.ds(i * 16, 16)
                glidx[sl] = glidx[sl] + node_l
                dd = gridx[sl]
                gridx[sl] = dd + node_r
                dv = dd + den_off
                lo = jnp.take_along_axis(dv, half, axis=0) * 2 + bit
                hi = jnp.take_along_axis(dv, half + 8, axis=0) * 2 + bit
                denidx[pl.ds(i * 32, 16)] = lo
                denidx[pl.ds(i * 32 + 16, 16)] = hi

            cl = pltpu.async_copy(tab.at[glidx], rows_l, sem_l)
            cr = pltpu.async_copy(tab.at[gridx], rows_r, sem_r)
            cl.wait()
            cr.wait()

            @pl.loop(0, _C)
            def _edge(e):
                p0 = jnp.zeros((16,), _f32)
                p1 = jnp.zeros((16,), _f32)
                for j in range(10):
                    sl = pl.ds(16 * j, 16)
                    t = rows_l[e, sl] + rows_r[e, sl]
                    t = jnp.maximum(t, 0.2 * t) * attregs[j]
                    if j < 5:
                        p0 = p0 + t
                    else:
                        p1 = p1 + t
                s0 = _hsum16(p0, lane)
                s1 = _hsum16(p1, lane)
                v = jnp.where(lane == 0, s0, s1)
                exv = jnp.exp(v)
                plsc.store_scatter(
                    exbuf, [2 * jnp.broadcast_to(e, (16,)).astype(_i32)
                            + lane],
                    exv, mask=lane < 2)

            pltpu.sync_copy(exbuf, den_sh.at[denidx], add=True)
            pltpu.sync_copy(
                exbuf, ex_out.at[pl.ds(2 * (ex_base + k * _C), 2 * _C)])

    plsc.subcore_barrier()

    for i in range(5):
        pltpu.sync_copy(den_sh.at[pl.ds(s * _DW + i * 1000, 1000)],
                        zstage.at[pl.ds(0, 1000)])
        pltpu.sync_copy(
            zstage.at[pl.ds(0, 1000)],
            den_out.at[pl.ds(2 * c * _BG + s * _DW + i * 1000, 1000)])


def _p1(tab, src, dst, att2):
    mesh = plsc.VectorSubcoreMesh(
        core_axis_name="c", subcore_axis_name="s",
        num_cores=_NC, num_subcores=_NS)
    return pl.kernel(
        _p1_body,
        out_type=(jax.ShapeDtypeStruct((2 * 2 * _B * _E,), _f32),
                  jax.ShapeDtypeStruct((2 * 2 * _BG,), _f32)),
        mesh=mesh,
        scratch_types=[
            pltpu.VMEM((_C, 160), _f32),
            pltpu.VMEM((_C, 160), _f32),
            pltpu.VMEM((_C,), _i32),
            pltpu.VMEM((_C,), _i32),
            pltpu.VMEM((2 * _C,), _f32),
            pltpu.VMEM((2 * _C,), _i32),
            pltpu.VMEM((1, 160), _f32),
            pltpu.VMEM((1024,), _f32),
            pltpu.VMEM_SHARED((2 * _BG,), _f32),
            pltpu.SemaphoreType.DMA,
            pltpu.SemaphoreType.DMA,
        ],
        compiler_params=pltpu.CompilerParams(
            needs_layout_passes=False, use_tc_tiling_on_sc=False),
    )(tab, src, dst, att2)


def _p2_body(tab, src_h, dst_h, ex_in, den_in, gat_out,
             rows, dstv, glidx, ex2, a0, a1, denidx, denbuf,
             out_sh, sem_g):
    c = lax.axis_index("c")
    s = lax.axis_index("s")
    lane = lax.iota(_i32, 16)
    half = lax.shift_right_logical(lane, 1)
    bit = lane & 1
    zero16 = lane * 0
    gl_off = (2 * c) * _BG

    zv = jnp.zeros((16,), _f32)

    @pl.loop(0, _B)
    def _batch(b):
        @pl.loop(0, _C)
        def _zfill(i):
            for j in range(10):
                rows[i, pl.ds(j * 16, 16)] = zv

        @pl.loop(s, _NCH, step=_NS)
        def _zero(k):
            pltpu.sync_copy(rows, out_sh.at[pl.ds(k * _C, _C)])

        plsc.subcore_barrier()

        node_l = gl_off + b * _G
        den_off = c * _BG + b * _G
        ex_base = (c * _B + b) * _E + s * _EPT

        @pl.loop(0, _NCH)
        def _chunk(k):
            eb = s * _EPT + k * _C
            pltpu.sync_copy(src_h.at[pl.ds(eb, _C)], glidx)
            pltpu.sync_copy(dst_h.at[pl.ds(eb, _C)], dstv)

            @pl.loop(0, _C // 16)
            def _ix(i):
                sl = pl.ds(i * 16, 16)
                glidx[sl] = glidx[sl] + node_l
                dv = dstv[sl] + den_off
                lo = jnp.take_along_axis(dv, half, axis=0) * 2 + bit
                hi = jnp.take_along_axis(dv, half + 8, axis=0) * 2 + bit
                denidx[pl.ds(i * 32, 16)] = lo
                denidx[pl.ds(i * 32 + 16, 16)] = hi

            cg = pltpu.async_copy(tab.at[glidx], rows, sem_g)
            pltpu.sync_copy(
                ex_in.at[pl.ds(2 * (ex_base + k * _C), 2 * _C)], ex2)
            pltpu.sync_copy(den_in.at[denidx], denbuf)
            cg.wait()

            @pl.loop(0, _C // 16)
            def _grp(g):
                sl = pl.ds(g * 16, 16)
                er2 = (zero16 + g * 16 + lane) * 2
                d0 = plsc.load_gather(denbuf, [er2])
                d1 = plsc.load_gather(denbuf, [er2 + 1])
                e0 = plsc.load_gather(ex2, [er2])
                e1 = plsc.load_gather(ex2, [er2 + 1])
                a0[sl] = e0 / (d0 + 1e-16)
                a1[sl] = e1 / (d1 + 1e-16)

            @pl.loop(0, _C)
            def _edge(e):
                eidx = jnp.broadcast_to(e, (16,)).astype(_i32)
                av0 = plsc.load_gather(a0, [eidx])
                av1 = plsc.load_gather(a1, [eidx])
                for j in range(5):
                    sl = pl.ds(16 * j, 16)
                    rows[e, sl] = rows[e, sl] * av0
                for j in range(5, 10):
                    sl = pl.ds(16 * j, 16)
                    rows[e, sl] = rows[e, sl] * av1

            pltpu.sync_copy(rows, out_sh.at[dstv], add=True)

        plsc.subcore_barrier()

        @pl.loop(s, _NCH, step=_NS)
        def _drain(k):
            pltpu.sync_copy(out_sh.at[pl.ds(k * _C, _C)], rows)
            pltpu.sync_copy(
                rows, gat_out.at[pl.ds(c * _BG + b * _G + k * _C, _C)])

        plsc.subcore_barrier()


def _p2(tab, src, dst, ex, den):
    mesh = plsc.VectorSubcoreMesh(
        core_axis_name="c", subcore_axis_name="s",
        num_cores=_NC, num_subcores=_NS)
    return pl.kernel(
        _p2_body,
        out_type=jax.ShapeDtypeStruct((2 * _BG, 160), _f32),
        mesh=mesh,
        scratch_types=[
            pltpu.VMEM((_C, 160), _f32),
            pltpu.VMEM((_C,), _i32),
            pltpu.VMEM((_C,), _i32),
            pltpu.VMEM((2 * _C,), _f32),
            pltpu.VMEM((_C,), _f32),
            pltpu.VMEM((_C,), _f32),
            pltpu.VMEM((2 * _C,), _i32),
            pltpu.VMEM((2 * _C,), _f32),
            pltpu.VMEM_SHARED((_G, 160), _f32),
            pltpu.SemaphoreType.DMA,
        ],
        compiler_params=pltpu.CompilerParams(
            needs_layout_passes=False, use_tc_tiling_on_sc=False),
    )(tab, src, dst, ex, den)


_TA = 1000
_NT = _G // _TA


def _a_common(x, lng, lnb, wq, wk, wv, projm, qp, ak, vout, kmax, b, t):
    xn = _ln(x, lng[0], lnb[0])
    q = jnp.dot(xn, wq[...], preferred_element_type=_f32)
    k = jnp.dot(xn, wk[...], preferred_element_type=_f32)
    v = jnp.dot(xn, wv[...], preferred_element_type=_f32)
    proj = projm[...]

    @pl.when(jnp.logical_and(b == 0, t == 0))
    def _():
        kmax[...] = jnp.full((1, 1), -jnp.inf, _f32)

    mx = kmax[0, 0]
    for h in range(_H):
        qh = q[:, h * _DH:(h + 1) * _DH] * _DN
        kh = k[:, h * _DH:(h + 1) * _DH] * _DN
        ddq = lax.dot_general(qh, proj, (((1,), (1,)), ((), ())),
                              preferred_element_type=_f32)
        ddk = lax.dot_general(kh, proj, (((1,), (1,)), ((), ())),
                              preferred_element_type=_f32)
        dq = 0.5 * jnp.sum(qh * qh, axis=-1, keepdims=True)
        dk = 0.5 * jnp.sum(kh * kh, axis=-1, keepdims=True)
        qp[0, h] = _RATIO * (
            jnp.exp(ddq - dq - jnp.max(ddq, axis=-1, keepdims=True)) + 1e-4)
        ak[0, h] = ddk - dk
        vout[0, h] = v[:, h * _DH:(h + 1) * _DH]
        mx = jnp.maximum(mx, jnp.max(ddk))
    kmax[...] = jnp.broadcast_to(mx, (1, 1))


def _a_body_first(g01, g23, gbias, lng, lnb, wq, wk, wv, projm,
                  x_out, qp, ak, vout, kmax):
    b = pl.program_id(0)
    t = pl.program_id(1)
    x = jnp.concatenate([g01[0], g23[0]], axis=-1) + gbias[0]
    x_out[0] = x
    _a_common(x, lng, lnb, wq, wk, wv, projm, qp, ak, vout, kmax, b, t)


def _a_body_next(x_ref, lng, lnb, wq, wk, wv, projm,
                 qp, ak, vout, kmax):
    b = pl.program_id(0)
    t = pl.program_id(1)
    _a_common(x_ref[0], lng, lnb, wq, wk, wv, projm, qp, ak, vout, kmax, b, t)


def _wspec(shape):
    n = len(shape)
    return pl.BlockSpec(shape, lambda b, t: (0,) * n)


_QP_OUT = [
    jax.ShapeDtypeStruct((_B, _H, _G, _M), _f32),
    jax.ShapeDtypeStruct((_B, _H, _G, _M), _f32),
    jax.ShapeDtypeStruct((_B, _H, _G, _DH), _f32),
    jax.ShapeDtypeStruct((1, 1), _f32),
]
_QP_SPECS = [
    pl.BlockSpec((1, _H, _TA, _M), lambda b, t: (b, 0, t, 0)),
    pl.BlockSpec((1, _H, _TA, _M), lambda b, t: (b, 0, t, 0)),
    pl.BlockSpec((1, _H, _TA, _DH), lambda b, t: (b, 0, t, 0)),
    pl.BlockSpec((1, 1), lambda b, t: (0, 0)),
]


def _a_first(g01, g23, gbias, lng, lnb, wq, wk, wv, projm):
    return pl.pallas_call(
        _a_body_first,
        grid=(_B, _NT),
        in_specs=[
            pl.BlockSpec((1, _TA, 160), lambda b, t: (b, t, 0)),
            pl.BlockSpec((1, _TA, 160), lambda b, t: (b, t, 0)),
            _wspec((1, _D)), _wspec((1, _D)), _wspec((1, _D)),
            _wspec((_D, _D)), _wspec((_D, _D)), _wspec((_D, _D)),
            _wspec((_M, _DH)),
        ],
        out_specs=[pl.BlockSpec((1, _TA, _D), lambda b, t: (b, t, 0))]
        + _QP_SPECS,
        out_shape=[jax.ShapeDtypeStruct((_B, _G, _D), _f32)] + _QP_OUT,
        compiler_params=pltpu.CompilerParams(
            dimension_semantics=("arbitrary", "arbitrary")),
    )(g01, g23, gbias, lng, lnb, wq, wk, wv, projm)


def _a_next(x, lng, lnb, wq, wk, wv, projm):
    return pl.pallas_call(
        _a_body_next,
        grid=(_B, _NT),
        in_specs=[
            pl.BlockSpec((1, _TA, _D), lambda b, t: (b, t, 0)),
            _wspec((1, _D)), _wspec((1, _D)),
            _wspec((_D, _D)), _wspec((_D, _D)), _wspec((_D, _D)),
            _wspec((_M, _DH)),
        ],
        out_specs=_QP_SPECS,
        out_shape=_QP_OUT,
        compiler_params=pltpu.CompilerParams(
            dimension_semantics=("arbitrary", "arbitrary")),
    )(x, lng, lnb, wq, wk, wv, projm)


def _b_body(ak, v, kmax, ctx, ksum):
    t = pl.program_id(1)
    km = kmax[0, 0]

    @pl.when(t == 0)
    def _():
        ctx[...] = jnp.zeros_like(ctx)
        ksum[...] = jnp.zeros_like(ksum)

    for h in range(_H):
        kp = _RATIO * (jnp.exp(ak[0, h] - km) + 1e-4)
        ctx[0, h] += lax.dot_general(kp, v[0, h], (((0,), (0,)), ((), ())),
                                     preferred_element_type=_f32)
        ksum[0, h] += jnp.sum(kp, axis=0, keepdims=True)


def _b(ak, v, kmax):
    return pl.pallas_call(
        _b_body,
        grid=(_B, _NT),
        in_specs=[
            pl.BlockSpec((1, _H, _TA, _M), lambda b, t: (b, 0, t, 0)),
            pl.BlockSpec((1, _H, _TA, _DH), lambda b, t: (b, 0, t, 0)),
            pl.BlockSpec((1, 1), lambda b, t: (0, 0)),
        ],
        out_specs=[
            pl.BlockSpec((1, _H, _M, _DH), lambda b, t: (b, 0, 0, 0)),
            pl.BlockSpec((1, _H, 1, _M), lambda b, t: (b, 0, 0, 0)),
        ],
        out_shape=[
            jax.ShapeDtypeStruct((_B, _H, _M, _DH), _f32),
            jax.ShapeDtypeStruct((_B, _H, 1, _M), _f32),
        ],
        compiler_params=pltpu.CompilerParams(
            dimension_semantics=("arbitrary", "arbitrary")),
    )(ak, v, kmax)


def _c_body_inner(qp, ctx, ksum, x_ref, wo, bo, ln2g, ln2b,
                  w1, b1, w2, b2, lnfg, lnfb, out, last):
    x = x_ref[0]
    os = []
    for h in range(_H):
        qph = qp[0, h]
        o = jnp.dot(qph, ctx[0, h], preferred_element_type=_f32)
        den = jnp.sum(qph * ksum[0, h], axis=1, keepdims=True)
        os.append(o / (den + 1e-6))
    o = jnp.concatenate(os, axis=-1)
    x2 = x + jnp.dot(o, wo[...], preferred_element_type=_f32) + bo[0]
    xn2 = _ln(x2, ln2g[0], ln2b[0])
    ff = jnp.dot(jax.nn.gelu(jnp.dot(xn2, w1[...],
                                     preferred_element_type=_f32) + b1[0]),
                 w2[...], preferred_element_type=_f32) + b2[0]
    x3 = x2 + ff
    out[0] = _ln(x3, lnfg[0], lnfb[0]) if last else x3


def _c(qp, ctx, ksum, x, wo, bo, ln2g, ln2b, w1, b1, w2, b2,
       lnfg, lnfb, last):
    body = functools.partial(_c_body_inner, last=last)
    return pl.pallas_call(
        body,
        grid=(_B, _NT),
        in_specs=[
            pl.BlockSpec((1, _H, _TA, _M), lambda b, t: (b, 0, t, 0)),
            pl.BlockSpec((1, _H, _M, _DH), lambda b, t: (b, 0, 0, 0)),
            pl.BlockSpec((1, _H, 1, _M), lambda b, t: (b, 0, 0, 0)),
            pl.BlockSpec((1, _TA, _D), lambda b, t: (b, t, 0)),
            _wspec((_D, _D)), _wspec((1, _D)),
            _wspec((1, _D)), _wspec((1, _D)),
            _wspec((_D, _FF)), _wspec((1, _FF)),
            _wspec((_FF, _D)), _wspec((1, _D)),
            _wspec((1, _D)), _wspec((1, _D)),
        ],
        out_specs=pl.BlockSpec((1, _TA, _D), lambda b, t: (b, t, 0)),
        out_shape=jax.ShapeDtypeStruct((_B, _G, _D), _f32),
        compiler_params=pltpu.CompilerParams(
            dimension_semantics=("arbitrary", "arbitrary")),
    )(qp, ctx, ksum, x, wo, bo, ln2g, ln2b, w1, b1, w2, b2, lnfg, lnfb)


def kernel(x, edge_index, gat_wl, gat_bl, gat_wr, gat_br, gat_att, gat_bias,
           ln1_g, ln1_b, wq, wk, wv, wo, bo, proj, ln2_g, ln2_b,
           ff_w1, ff_b1, ff_w2, ff_b2, lnf_g, lnf_b):
    src = edge_index[0].astype(_i32)
    dst = edge_index[1].astype(_i32)
    xf = x.reshape(_BG, _D)

    wlr = jnp.concatenate([gat_wl, gat_wr], axis=1)
    blr = jnp.concatenate([gat_bl, gat_br]).reshape(1, 2 * _D)
    tab4 = _k0(xf, wlr, blr)
    tab = tab4.reshape(4 * _BG, 160)

    att2 = gat_att.reshape(2, 1, 160)

    ex, den = _p1(tab, src, dst, att2)
    gat = _p2(tab, src, dst, ex, den)

    g01 = gat[:_BG].reshape(_B, _G, 160)
    g23 = gat[_BG:].reshape(_B, _G, 160)

    gbias = gat_bias.reshape(1, _D)
    xcur = None
    for l in range(_L):
        lng = ln1_g[l].reshape(1, _D)
        lnb = ln1_b[l].reshape(1, _D)
        if l == 0:
            xcur, qp, ak, v, kmax = _a_first(
                g01, g23, gbias, lng, lnb, wq[l], wk[l], wv[l], proj[l])
        else:
            qp, ak, v, kmax = _a_next(
                xcur, lng, lnb, wq[l], wk[l], wv[l], proj[l])
        ctx, ksum = _b(ak, v, kmax)
        xcur = _c(qp, ctx, ksum, xcur,
                  wo[l], bo[l].reshape(1, _D),
                  ln2_g[l].reshape(1, _D), ln2_b[l].reshape(1, _D),
                  ff_w1[l], ff_b1[l].reshape(1, _FF),
                  ff_w2[l], ff_b2[l].reshape(1, _D),
                  lnf_g.reshape(1, _D), lnf_b.reshape(1, _D),
                  last=(l == _L - 1))
    return xcur


# speedup vs baseline: 17.7170x; 1.4497x over previous
import functools

import jax
import jax.numpy as jnp
from jax import lax
from jax.experimental import pallas as pl
from jax.experimental.pallas import tpu as pltpu
from jax.experimental.pallas import tpu_sc as plsc

_B, _G, _D, _H = 4, 10000, 320, 4
_DH = _D // _H
_E = 160000
_M = 256
_FF = 4 * _D
_L = 2
_BG = _B * _G
_RATIO = _M ** -0.5
_DN = _DH ** -0.25

_NC, _NS = 2, 16
_EPT = _E // _NS
_C = 80
_NCH = _EPT // _C
_DW = 2 * _BG // _NS

_f32 = jnp.float32
_i32 = jnp.int32


def _hsum16(v, lane):
    for k in (8, 4, 2, 1):
        v = v + jnp.take_along_axis(v, lane ^ k, axis=0)
    return v


def _ln(x, g, b):
    mu = jnp.mean(x, axis=-1, keepdims=True)
    xc = x - mu
    var = jnp.mean(xc * xc, axis=-1, keepdims=True)
    return xc * lax.rsqrt(var + 1e-5) * g + b


def _k0_body(x_ref, w_ref, b_ref, o_ref):
    res = jnp.dot(x_ref[...], w_ref[...], preferred_element_type=_f32)
    res = res + b_ref[0]
    o_ref[0] = res[:, 0:160]
    o_ref[1] = res[:, 320:480]
    o_ref[2] = res[:, 160:320]
    o_ref[3] = res[:, 480:640]


def _k0(xf, wlr, blr):
    t0 = 2000
    return pl.pallas_call(
        _k0_body,
        grid=(_BG // t0,),
        in_specs=[
            pl.BlockSpec((t0, _D), lambda t: (t, 0)),
            pl.BlockSpec((_D, 2 * _D), lambda t: (0, 0)),
            pl.BlockSpec((1, 2 * _D), lambda t: (0, 0)),
        ],
        out_specs=pl.BlockSpec((4, t0, 160), lambda t: (0, t, 0)),
        out_shape=jax.ShapeDtypeStruct((4, _BG, 160), _f32),
        compiler_params=pltpu.CompilerParams(
            dimension_semantics=("arbitrary",)),
    )(xf, wlr, blr)


_TOT1 = _B * _NCH


def _edge_loop_p1(rl, rr, exb, attregs, lane):
    @pl.loop(0, _C)
    def _edge(e):
        p0 = jnp.zeros((16,), _f32)
        p1 = jnp.zeros((16,), _f32)
        for j in range(10):
            sl = pl.ds(16 * j, 16)
            t = rl[e, sl] + rr[e, sl]
            t = jnp.maximum(t, 0.2 * t) * attregs[j]
            if j < 5:
                p0 = p0 + t
            else:
                p1 = p1 + t
        s0 = _hsum16(p0, lane)
        s1 = _hsum16(p1, lane)
        v = jnp.where(lane == 0, s0, s1)
        plsc.store_scatter(
            exb, [2 * jnp.broadcast_to(e, (16,)).astype(_i32) + lane],
            jnp.exp(v), mask=lane < 2)


def _p1_body(tab, eidx, att2, ex_out, den_out,
             rows_l0, rows_l1, rows_r0, rows_r1,
             idx0, idx1, gl0, gl1, gr0, gr1,
             dix0, dix1, ex0, ex1, attv, zstage, den_sh,
             s_i0, s_i1, s_gl0, s_gl1, s_gr0, s_gr1,
             s_d0, s_d1, s_e0, s_e1):
    c = lax.axis_index("c")
    s = lax.axis_index("s")
    lane = lax.iota(_i32, 16)
    half = lax.shift_right_logical(lane, 1)
    bit = lane & 1
    gl_off = (2 * c) * _BG
    gr_off = gl_off + _BG
    zv = jnp.zeros((16,), _f32)

    @pl.loop(0, 64)
    def _zfill(i):
        zstage[pl.ds(i * 16, 16)] = zv

    for i in range(5):
        pltpu.sync_copy(zstage.at[pl.ds(0, 1000)],
                        den_sh.at[pl.ds(s * _DW + i * 1000, 1000)])

    pltpu.sync_copy(att2.at[c], attv)
    plsc.subcore_barrier()

    attregs = [attv[0, pl.ds(16 * j, 16)] for j in range(10)]

    sets = ((rows_l0, rows_r0, idx0, gl0, gr0, dix0, ex0,
             s_i0, s_gl0, s_gr0, s_d0, s_e0),
            (rows_l1, rows_r1, idx1, gl1, gr1, dix1, ex1,
             s_i1, s_gl1, s_gr1, s_d1, s_e1))

    def issue_idx(cn, p):
        rl, rr, ix, gl, gr, dix, exb, si, sgl, sgr, sd, se = sets[p]
        b = cn // _NCH
        k = cn - b * _NCH
        eb = s * _EPT + k * _C
        pltpu.async_copy(eidx.at[:, pl.ds(eb, _C)], ix, si)

    def prep_issue_gather(cn, p):
        rl, rr, ix, gl, gr, dix, exb, si, sgl, sgr, sd, se = sets[p]
        pltpu.make_async_copy(eidx.at[:, pl.ds(0, _C)], ix, si).wait()
        b = cn // _NCH
        node_l = gl_off + b * _G
        node_r = gr_off + b * _G

        @pl.loop(0, _C // 16)
        def _ix(i):
            sl = pl.ds(i * 16, 16)
            gl[sl] = ix[0, sl] + node_l
            gr[sl] = ix[1, sl] + node_r

        pltpu.async_copy(tab.at[gl], rl, sgl)
        pltpu.async_copy(tab.at[gr], rr, sgr)

    def wait_gathers(p):
        rl, rr, ix, gl, gr, dix, exb, si, sgl, sgr, sd, se = sets[p]
        pltpu.make_async_copy(tab.at[pl.ds(0, _C)], rl, sgl).wait()
        pltpu.make_async_copy(tab.at[pl.ds(0, _C)], rr, sgr).wait()

    def wait_stores(p):
        rl, rr, ix, gl, gr, dix, exb, si, sgl, sgr, sd, se = sets[p]
        pltpu.make_async_copy(exb, den_sh.at[pl.ds(0, 2 * _C)], sd).wait()
        pltpu.make_async_copy(exb, ex_out.at[pl.ds(0, 2 * _C)], se).wait()

    def build_denidx(cn, p):
        rl, rr, ix, gl, gr, dix, exb, si, sgl, sgr, sd, se = sets[p]
        b = cn // _NCH
        den_off = b * _G

        @pl.loop(0, _C // 16)
        def _dx(i):
            dv = ix[1, pl.ds(i * 16, 16)] + den_off
            lo = jnp.take_along_axis(dv, half, axis=0) * 2 + bit
            hi = jnp.take_along_axis(dv, half + 8, axis=0) * 2 + bit
            dix[pl.ds(i * 32, 16)] = lo
            dix[pl.ds(i * 32 + 16, 16)] = hi

    def compute_store(cn, p):
        rl, rr, ix, gl, gr, dix, exb, si, sgl, sgr, sd, se = sets[p]
        _edge_loop_p1(rl, rr, exb, attregs, lane)
        b = cn // _NCH
        k = cn - b * _NCH
        ex_base = (c * _B + b) * _E + s * _EPT + k * _C
        pltpu.async_copy(exb, den_sh.at[dix], sd, add=True)
        pltpu.async_copy(exb, ex_out.at[pl.ds(2 * ex_base, 2 * _C)], se)

    issue_idx(0, 0)
    issue_idx(1, 1)
    prep_issue_gather(0, 0)
    prep_issue_gather(1, 1)

    @pl.loop(0, _TOT1, step=2)
    def _main(kk):
        for off in (0, 1):
            cn = kk + off
            p = off
            wait_gathers(p)

            @pl.when(cn >= 2)
            def _ws():
                wait_stores(p)

            build_denidx(cn, p)

            @pl.when(cn + 2 < _TOT1)
            def _ii():
                issue_idx(cn + 2, p)

            compute_store(cn, p)

            @pl.when(cn + 2 < _TOT1)
            def _pg():
                prep_issue_gather(cn + 2, p)

    wait_stores(0)
    wait_stores(1)
    plsc.subcore_barrier()

    for i in range(5):
        pltpu.sync_copy(den_sh.at[pl.ds(s * _DW + i * 1000, 1000)],
                        zstage.at[pl.ds(0, 1000)])
        pltpu.sync_copy(
            zstage.at[pl.ds(0, 1000)],
            den_out.at[pl.ds(2 * c * _BG + s * _DW + i * 1000, 1000)])


def _p1(tab, eidx, att2):
    mesh = plsc.VectorSubcoreMesh(
        core_axis_name="c", subcore_axis_name="s",
        num_cores=_NC, num_subcores=_NS)
    return pl.kernel(
        _p1_body,
        out_type=(jax.ShapeDtypeStruct((2 * 2 * _B * _E,), _f32),
                  jax.ShapeDtypeStruct((2 * 2 * _BG,), _f32)),
        mesh=mesh,
        scratch_types=[
            pltpu.VMEM((_C, 160), _f32),
            pltpu.VMEM((_C, 160), _f32),
            pltpu.VMEM((_C, 160), _f32),
            pltpu.VMEM((_C, 160), _f32),
            pltpu.VMEM((2, _C), _i32),
            pltpu.VMEM((2, _C), _i32),
            pltpu.VMEM((_C,), _i32),
            pltpu.VMEM((_C,), _i32),
            pltpu.VMEM((_C,), _i32),
            pltpu.VMEM((_C,), _i32),
            pltpu.VMEM((2 * _C,), _i32),
            pltpu.VMEM((2 * _C,), _i32),
            pltpu.VMEM((2 * _C,), _f32),
            pltpu.VMEM((2 * _C,), _f32),
            pltpu.VMEM((1, 160), _f32),
            pltpu.VMEM((1024,), _f32),
            pltpu.VMEM_SHARED((2 * _BG,), _f32),
        ] + [pltpu.SemaphoreType.DMA] * 10,
        compiler_params=pltpu.CompilerParams(
            needs_layout_passes=False, use_tc_tiling_on_sc=False),
    )(tab, eidx, att2)


def _p2_body(tab, eidx, ex_in, den_in, gat_out,
             rows0, rows1, idx0, idx1, gl0, gl1,
             dix0, dix1, db0, db1, exn0, exn1, a0, a1,
             out_sh,
             s_i0, s_i1, s_g0, s_g1, s_x0, s_x1, s_d0, s_d1):
    c = lax.axis_index("c")
    s = lax.axis_index("s")
    lane = lax.iota(_i32, 16)
    half = lax.shift_right_logical(lane, 1)
    bit = lane & 1
    zero16 = lane * 0
    gl_off = (2 * c) * _BG
    zv = jnp.zeros((16,), _f32)

    sets = ((rows0, idx0, gl0, dix0, db0, exn0, s_i0, s_g0, s_x0, s_d0),
            (rows1, idx1, gl1, dix1, db1, exn1, s_i1, s_g1, s_x1, s_d1))

    def issue_idx(cn, p):
        rw, ix, gl, dix, db, exn, si, sg, sx, sd = sets[p]
        eb = s * _EPT + cn * _C
        pltpu.async_copy(eidx.at[:, pl.ds(eb, _C)], ix, si)

    def prep_issue_reads(b, cn, p):
        rw, ix, gl, dix, db, exn, si, sg, sx, sd = sets[p]
        pltpu.make_async_copy(eidx.at[:, pl.ds(0, _C)], ix, si).wait()
        node_l = gl_off + b * _G
        den_off = c * _BG + b * _G

        @pl.loop(0, _C // 16)
        def _ix(i):
            sl = pl.ds(i * 16, 16)
            gl[sl] = ix[0, sl] + node_l
            dv = ix[1, sl] + den_off
            lo = jnp.take_along_axis(dv, half, axis=0) * 2 + bit
            hi = jnp.take_along_axis(dv, half + 8, axis=0) * 2 + bit
            dix[pl.ds(i * 32, 16)] = lo
            dix[pl.ds(i * 32 + 16, 16)] = hi

        pltpu.async_copy(tab.at[gl], rw, sg)
        ex_base = (c * _B + b) * _E + s * _EPT + cn * _C
        pltpu.async_copy(ex_in.at[pl.ds(2 * ex_base, 2 * _C)], exn, sx)
        pltpu.async_copy(den_in.at[dix], db, sd)

    def wait_reads(p):
        rw, ix, gl, dix, db, exn, si, sg, sx, sd = sets[p]
        pltpu.make_async_copy(tab.at[pl.ds(0, _C)], rw, sg).wait()
        pltpu.make_async_copy(ex_in.at[pl.ds(0, 2 * _C)], exn, sx).wait()
        pltpu.make_async_copy(ex_in.at[pl.ds(0, 2 * _C)], db, sd).wait()

    def compute_scatter(p):
        rw, ix, gl, dix, db, exn, si, sg, sx, sd = sets[p]

        @pl.loop(0, _C // 16)
        def _grp(g):
            sl = pl.ds(g * 16, 16)
            er2 = (zero16 + g * 16 + lane) * 2
            d0 = plsc.load_gather(db, [er2])
            d1 = plsc.load_gather(db, [er2 + 1])
            e0 = plsc.load_gather(exn, [er2])
            e1 = plsc.load_gather(exn, [er2 + 1])
            a0[sl] = e0 / (d0 + 1e-16)
            a1[sl] = e1 / (d1 + 1e-16)

        @pl.loop(0, _C)
        def _edge(e):
            ei = jnp.broadcast_to(e, (16,)).astype(_i32)
            av0 = plsc.load_gather(a0, [ei])
            av1 = plsc.load_gather(a1, [ei])
            for j in range(5):
                sl = pl.ds(16 * j, 16)
                rw[e, sl] = rw[e, sl] * av0
            for j in range(5, 10):
                sl = pl.ds(16 * j, 16)
                rw[e, sl] = rw[e, sl] * av1

        pltpu.sync_copy(rw, out_sh.at[ix.at[1]], add=True)

    @pl.loop(0, _B)
    def _batch(b):
        @pl.loop(0, _C)
        def _zfill(i):
            for j in range(10):
                rows0[i, pl.ds(j * 16, 16)] = zv

        @pl.loop(s, _NCH, step=_NS)
        def _zero(k):
            pltpu.sync_copy(rows0, out_sh.at[pl.ds(k * _C, _C)])

        plsc.subcore_barrier()

        issue_idx(0, 0)
        issue_idx(1, 1)
        prep_issue_reads(b, 0, 0)
        prep_issue_reads(b, 1, 1)

        @pl.loop(0, _NCH - 1, step=2)
        def _main(kk):
            for off in (0, 1):
                cn = kk + off
                p = off
                wait_reads(p)
                compute_scatter(p)

                @pl.when(cn + 2 < _NCH)
                def _nxt():
                    issue_idx(cn + 2, p)
                    prep_issue_reads(b, cn + 2, p)

        wait_reads(0)
        compute_scatter(0)

        plsc.subcore_barrier()

        @pl.loop(s, _NCH, step=_NS)
        def _drain(k):
            pltpu.sync_copy(out_sh.at[pl.ds(k * _C, _C)], rows0)
            pltpu.sync_copy(
                rows0, gat_out.at[pl.ds(c * _BG + b * _G + k * _C, _C)])

        plsc.subcore_barrier()


def _p2(tab, eidx, ex, den):
    mesh = plsc.VectorSubcoreMesh(
        core_axis_name="c", subcore_axis_name="s",
        num_cores=_NC, num_subcores=_NS)
    return pl.kernel(
        _p2_body,
        out_type=jax.ShapeDtypeStruct((2 * _BG, 160), _f32),
        mesh=mesh,
        scratch_types=[
            pltpu.VMEM((_C, 160), _f32),
            pltpu.VMEM((_C, 160), _f32),
            pltpu.VMEM((2, _C), _i32),
            pltpu.VMEM((2, _C), _i32),
            pltpu.VMEM((_C,), _i32),
            pltpu.VMEM((_C,), _i32),
            pltpu.VMEM((2 * _C,), _i32),
            pltpu.VMEM((2 * _C,), _i32),
            pltpu.VMEM((2 * _C,), _f32),
            pltpu.VMEM((2 * _C,), _f32),
            pltpu.VMEM((2 * _C,), _f32),
            pltpu.VMEM((2 * _C,), _f32),
            pltpu.VMEM((_C,), _f32),
            pltpu.VMEM((_C,), _f32),
            pltpu.VMEM_SHARED((_G, 160), _f32),
        ] + [pltpu.SemaphoreType.DMA] * 8,
        compiler_params=pltpu.CompilerParams(
            needs_layout_passes=False, use_tc_tiling_on_sc=False),
    )(tab, eidx, ex, den)


_TA = 1000
_NT = _G // _TA


def _a_common(x, lng, lnb, wq, wk, wv, projm, qp, ak, vout, kmax, b, t):
    xn = _ln(x, lng[0], lnb[0])
    q = jnp.dot(xn, wq[...], preferred_element_type=_f32) * _DN
    k = jnp.dot(xn, wk[...], preferred_element_type=_f32) * _DN
    v = jnp.dot(xn, wv[...], preferred_element_type=_f32)
    proj = projm[...]

    @pl.when(jnp.logical_and(b == 0, t == 0))
    def _():
        kmax[...] = jnp.full((1, 1), -jnp.inf, _f32)

    mx = kmax[0, 0]
    for h in range(_H):
        qh = q[:, h * _DH:(h + 1) * _DH]
        kh = k[:, h * _DH:(h + 1) * _DH]
        ddq = lax.dot_general(qh, proj, (((1,), (1,)), ((), ())),
                              preferred_element_type=_f32)
        ddk = lax.dot_general(kh, proj, (((1,), (1,)), ((), ())),
                              preferred_element_type=_f32)
        dq = 0.5 * jnp.sum(qh * qh, axis=-1, keepdims=True)
        dk = 0.5 * jnp.sum(kh * kh, axis=-1, keepdims=True)
        qp[0, h] = _RATIO * (
            jnp.exp(ddq - dq - jnp.max(ddq, axis=-1, keepdims=True)) + 1e-4)
        ak[0, h] = ddk - dk
        vout[0, h] = v[:, h * _DH:(h + 1) * _DH]
        mx = jnp.maximum(mx, jnp.max(ddk))
    kmax[...] = jnp.broadcast_to(mx, (1, 1))


def _a_body_first(g01, g23, gbias, lng, lnb, wq, wk, wv, projm,
                  x_out, qp, ak, vout, kmax):
    b = pl.program_id(0)
    t = pl.program_id(1)
    x = jnp.concatenate([g01[0], g23[0]], axis=-1) + gbias[0]
    x_out[0] = x
    _a_common(x, lng, lnb, wq, wk, wv, projm, qp, ak, vout, kmax, b, t)


def _a_body_next(x_ref, lng, lnb, wq, wk, wv, projm,
                 qp, ak, vout, kmax):
    b = pl.program_id(0)
    t = pl.program_id(1)
    _a_common(x_ref[0], lng, lnb, wq, wk, wv, projm, qp, ak, vout, kmax, b, t)


def _wspec(shape):
    n = len(shape)
    return pl.BlockSpec(shape, lambda b, t: (0,) * n)


_QP_OUT = [
    jax.ShapeDtypeStruct((_B, _H, _G, _M), _f32),
    jax.ShapeDtypeStruct((_B, _H, _G, _M), _f32),
    jax.ShapeDtypeStruct((_B, _H, _G, _DH), _f32),
    jax.ShapeDtypeStruct((1, 1), _f32),
]
_QP_SPECS = [
    pl.BlockSpec((1, _H, _TA, _M), lambda b, t: (b, 0, t, 0)),
    pl.BlockSpec((1, _H, _TA, _M), lambda b, t: (b, 0, t, 0)),
    pl.BlockSpec((1, _H, _TA, _DH), lambda b, t: (b, 0, t, 0)),
    pl.BlockSpec((1, 1), lambda b, t: (0, 0)),
]


def _a_first(g01, g23, gbias, lng, lnb, wq, wk, wv, projm):
    return pl.pallas_call(
        _a_body_first,
        grid=(_B, _NT),
        in_specs=[
            pl.BlockSpec((1, _TA, 160), lambda b, t: (b, t, 0)),
            pl.BlockSpec((1, _TA, 160), lambda b, t: (b, t, 0)),
            _wspec((1, _D)), _wspec((1, _D)), _wspec((1, _D)),
            _wspec((_D, _D)), _wspec((_D, _D)), _wspec((_D, _D)),
            _wspec((_M, _DH)),
        ],
        out_specs=[pl.BlockSpec((1, _TA, _D), lambda b, t: (b, t, 0))]
        + _QP_SPECS,
        out_shape=[jax.ShapeDtypeStruct((_B, _G, _D), _f32)] + _QP_OUT,
        compiler_params=pltpu.CompilerParams(
            dimension_semantics=("arbitrary", "arbitrary")),
    )(g01, g23, gbias, lng, lnb, wq, wk, wv, projm)


def _a_next(x, lng, lnb, wq, wk, wv, projm):
    return pl.pallas_call(
        _a_body_next,
        grid=(_B, _NT),
        in_specs=[
            pl.BlockSpec((1, _TA, _D), lambda b, t: (b, t, 0)),
            _wspec((1, _D)), _wspec((1, _D)),
            _wspec((_D, _D)), _wspec((_D, _D)), _wspec((_D, _D)),
            _wspec((_M, _DH)),
        ],
        out_specs=_QP_SPECS,
        out_shape=_QP_OUT,
        compiler_params=pltpu.CompilerParams(
            dimension_semantics=("arbitrary", "arbitrary")),
    )(x, lng, lnb, wq, wk, wv, projm)


def _b_body(ak, v, kmax, ctx, ksum):
    t = pl.program_id(1)
    km = kmax[0, 0]

    @pl.when(t == 0)
    def _():
        ctx[...] = jnp.zeros_like(ctx)
        ksum[...] = jnp.zeros_like(ksum)

    for h in range(_H):
        kp = _RATIO * (jnp.exp(ak[0, h] - km) + 1e-4)
        ctx[0, h] += lax.dot_general(kp, v[0, h], (((0,), (0,)), ((), ())),
                                     preferred_element_type=_f32)
        ksum[0, h] += jnp.sum(kp, axis=0, keepdims=True)


def _b(ak, v, kmax):
    return pl.pallas_call(
        _b_body,
        grid=(_B, _NT),
        in_specs=[
            pl.BlockSpec((1, _H, _TA, _M), lambda b, t: (b, 0, t, 0)),
            pl.BlockSpec((1, _H, _TA, _DH), lambda b, t: (b, 0, t, 0)),
            pl.BlockSpec((1, 1), lambda b, t: (0, 0)),
        ],
        out_specs=[
            pl.BlockSpec((1, _H, _M, _DH), lambda b, t: (b, 0, 0, 0)),
            pl.BlockSpec((1, _H, 1, _M), lambda b, t: (b, 0, 0, 0)),
        ],
        out_shape=[
            jax.ShapeDtypeStruct((_B, _H, _M, _DH), _f32),
            jax.ShapeDtypeStruct((_B, _H, 1, _M), _f32),
        ],
        compiler_params=pltpu.CompilerParams(
            dimension_semantics=("arbitrary", "arbitrary")),
    )(ak, v, kmax)


def _c_body_inner(qp, ctx, ksum, x_ref, wo, bo, ln2g, ln2b,
                  w1, b1, w2, b2, lnfg, lnfb, out, last):
    x = x_ref[0]
    os = []
    for h in range(_H):
        qph = qp[0, h]
        o = jnp.dot(qph, ctx[0, h], preferred_element_type=_f32)
        den = jnp.sum(qph * ksum[0, h], axis=1, keepdims=True)
        os.append(o / (den + 1e-6))
    o = jnp.concatenate(os, axis=-1)
    x2 = x + jnp.dot(o, wo[...], preferred_element_type=_f32) + bo[0]
    xn2 = _ln(x2, ln2g[0], ln2b[0])
    ff = jnp.dot(jax.nn.gelu(jnp.dot(xn2, w1[...],
                                     preferred_element_type=_f32) + b1[0]),
                 w2[...], preferred_element_type=_f32) + b2[0]
    x3 = x2 + ff
    out[0] = _ln(x3, lnfg[0], lnfb[0]) if last else x3


def _c(qp, ctx, ksum, x, wo, bo, ln2g, ln2b, w1, b1, w2, b2,
       lnfg, lnfb, last):
    body = functools.partial(_c_body_inner, last=last)
    return pl.pallas_call(
        body,
        grid=(_B, _NT),
        in_specs=[
            pl.BlockSpec((1, _H, _TA, _M), lambda b, t: (b, 0, t, 0)),
            pl.BlockSpec((1, _H, _M, _DH), lambda b, t: (b, 0, 0, 0)),
            pl.BlockSpec((1, _H, 1, _M), lambda b, t: (b, 0, 0, 0)),
            pl.BlockSpec((1, _TA, _D), lambda b, t: (b, t, 0)),
            _wspec((_D, _D)), _wspec((1, _D)),
            _wspec((1, _D)), _wspec((1, _D)),
            _wspec((_D, _FF)), _wspec((1, _FF)),
            _wspec((_FF, _D)), _wspec((1, _D)),
            _wspec((1, _D)), _wspec((1, _D)),
        ],
        out_specs=pl.BlockSpec((1, _TA, _D), lambda b, t: (b, t, 0)),
        out_shape=jax.ShapeDtypeStruct((_B, _G, _D), _f32),
        compiler_params=pltpu.CompilerParams(
            dimension_semantics=("arbitrary", "arbitrary")),
    )(qp, ctx, ksum, x, wo, bo, ln2g, ln2b, w1, b1, w2, b2, lnfg, lnfb)


def kernel(x, edge_index, gat_wl, gat_bl, gat_wr, gat_br, gat_att, gat_bias,
           ln1_g, ln1_b, wq, wk, wv, wo, bo, proj, ln2_g, ln2_b,
           ff_w1, ff_b1, ff_w2, ff_b2, lnf_g, lnf_b):
    eidx = edge_index.astype(_i32)
    xf = x.reshape(_BG, _D)

    wlr = jnp.concatenate([gat_wl, gat_wr], axis=1)
    blr = jnp.concatenate([gat_bl, gat_br]).reshape(1, 2 * _D)
    tab4 = _k0(xf, wlr, blr)
    tab = tab4.reshape(4 * _BG, 160)

    att2 = gat_att.reshape(2, 1, 160)

    ex, den = _p1(tab, eidx, att2)
    gat = _p2(tab, eidx, ex, den)

    g01 = gat[:_BG].reshape(_B, _G, 160)
    g23 = gat[_BG:].reshape(_B, _G, 160)

    gbias = gat_bias.reshape(1, _D)
    xcur = None
    for l in range(_L):
        lng = ln1_g[l].reshape(1, _D)
        lnb = ln1_b[l].reshape(1, _D)
        if l == 0:
            xcur, qp, ak, v, kmax = _a_first(
                g01, g23, gbias, lng, lnb, wq[l], wk[l], wv[l], proj[l])
        else:
            qp, ak, v, kmax = _a_next(
                xcur, lng, lnb, wq[l], wk[l], wv[l], proj[l])
        ctx, ksum = _b(ak, v, kmax)
        xcur = _c(qp, ctx, ksum, xcur,
                  wo[l], bo[l].reshape(1, _D),
                  ln2_g[l].reshape(1, _D), ln2_b[l].reshape(1, _D),
                  ff_w1[l], ff_b1[l].reshape(1, _FF),
                  ff_w2[l], ff_b2[l].reshape(1, _D),
                  lnf_g.reshape(1, _D), lnf_b.reshape(1, _D),
                  last=(l == _L - 1))
    return xcur
